# baseline (speedup 1.0000x reference)
import functools

import jax
import jax.numpy as jnp
from jax import lax
from jax.experimental import pallas as pl
from jax.experimental.pallas import tpu as pltpu
from jax.experimental.pallas import tpu_sc as plsc

_BATCH = 4096
_HIST = 200
_EMBED = 32
_NC = 2
_NS = 16
_NW = _NC * _NS
_NBLK = 1600
_BPW = _NBLK // _NW
_SLAB = 4


def _build():
    mesh = plsc.VectorSubcoreMesh(core_axis_name="c", subcore_axis_name="s")

    buf = lambda shape, dt: [pltpu.VMEM(shape, dt) for _ in range(2)]
    sems = lambda n: [pltpu.SemaphoreType.DMA for _ in range(n)]

    @functools.partial(
        pl.kernel,
        mesh=mesh,
        out_type=jax.ShapeDtypeStruct((_HIST, 4, 32, 8, 128), jnp.float32),
        scratch_types=[
            buf((_SLAB, 128), jnp.int32),
            buf((_SLAB * 128, _EMBED), jnp.float32),
            buf((_SLAB * 128 * 33,), jnp.float32),
            buf((4, _SLAB, 8, 128), jnp.float32),
            sems(2), sems(2), sems(2),
        ],
        compiler_params=pltpu.CompilerParams(
            use_tc_tiling_on_sc=False, needs_layout_passes=False),
    )
    def gather_kernel(idx_hbm, table_hbm, out_hbm, idx_v, rows_v, pad_v,
                      out_v, isem, gsem, osem):
        w = lax.axis_index("s") * _NC + lax.axis_index("c")
        iota = lax.iota(jnp.int32, 16)
        iota33 = iota * 33

        def blk_coords(bid):
            h = bid // 8
            tb0 = (bid % 8) * _SLAB
            return h, h // 8, h % 8, tb0

        def fire_idx(bid, p, sync):
            h, th, k, tb0 = blk_coords(bid)
            for j in range(_SLAB):
                if sync:
                    pltpu.sync_copy(idx_hbm.at[th, tb0 + j, k], idx_v[p].at[j])
                else:
                    pltpu.make_async_copy(
                        idx_hbm.at[th, tb0 + j, k], idx_v[p].at[j],
                        isem[p]).start()

        def drain_idx(p):
            for j in range(_SLAB):
                pltpu.make_async_copy(
                    idx_hbm.at[0, j, 0], idx_v[p].at[j], isem[p]).wait()

        def fire_gather(p):
            for j in range(_SLAB):
                pltpu.make_async_copy(
                    table_hbm.at[idx_v[p].at[j]],
                    rows_v[p].at[pl.ds(j * 128, 128)],
                    gsem[p]).start()

        def drain_gather(p):
            for j in range(_SLAB):
                pltpu.make_async_copy(
                    table_hbm.at[idx_v[p].at[j]],
                    rows_v[p].at[pl.ds(j * 128, 128)],
                    gsem[p]).wait()

        def fire_out(bid, p):
            h, th, k, tb0 = blk_coords(bid)
            for i in range(4):
                pltpu.make_async_copy(
                    out_v[p].at[i], out_hbm.at[h, i, pl.ds(tb0, _SLAB)],
                    osem[p]).start()

        def drain_out(p):
            for i in range(4):
                pltpu.make_async_copy(
                    out_v[p].at[i], out_hbm.at[0, i, pl.ds(0, _SLAB)],
                    osem[p]).wait()

        def transpose(p):
            def abody(r0, carry):
                for rr in range(4):
                    r = r0 * 4 + rr
                    base = r * 33
                    pad_v[p][pl.ds(base, 16)] = rows_v[p][r, pl.ds(0, 16)]
                    pad_v[p][pl.ds(base + 16, 16)] = rows_v[p][r, pl.ds(16, 16)]
                return carry
            lax.fori_loop(0, 128, abody, 0)

            def tbody(t, carry):
                k2 = t // 8
                lbase = (t % 8) * 16
                for j in range(_SLAB):
                    base = (j * 128 + lbase) * 33 + k2
                    for i in range(4):
                        ids = (base + 8 * i) + iota33
                        vals = plsc.load_gather(pad_v[p], [ids])
                        out_v[p][i, j, k2, pl.ds(lbase, 16)] = vals
                return carry
            lax.fori_loop(0, 64, tbody, 0)

        for p in range(2):
            fire_idx(w * _BPW + p, p, sync=True)
            fire_gather(p)

        def body(g, carry):
            for p in range(2):
                bid = w * _BPW + 2 * g + p
                nxt = bid + 2
                drain_gather(p)

                @pl.when(2 * g + p + 2 < _BPW)
                def _():
                    fire_idx(nxt, p, sync=False)

                @pl.when(g > 0)
                def _():
                    drain_out(p)

                transpose(p)
                fire_out(bid, p)

                @pl.when(2 * g + p + 2 < _BPW)
                def _():
                    drain_idx(p)
                    fire_gather(p)
            return carry

        lax.fori_loop(0, _BPW // 2, body, 0)
        for p in range(2):
            drain_out(p)

    return gather_kernel


_gather = _build()


@jax.jit
def kernel(input_variable, table):
    idx_n = (input_variable.astype(jnp.int32).T
             .reshape(25, 8, 32, 128).transpose(0, 2, 1, 3))
    n = _gather(idx_n, table)
    return n.transpose(2, 4, 0, 1, 3).reshape(_BATCH, _HIST, _EMBED)


# speedup vs baseline: 1.5383x; 1.5383x over previous
import functools

import jax
import jax.numpy as jnp
from jax import lax
from jax.experimental import pallas as pl
from jax.experimental.pallas import tpu as pltpu
from jax.experimental.pallas import tpu_sc as plsc

_BATCH = 4096
_HIST = 200
_EMBED = 32
_NC = 2
_NS = 16
_NW = _NC * _NS
_NBLK = 1600
_BPW = _NBLK // _NW
_SLAB = 4


def _build():
    mesh = plsc.VectorSubcoreMesh(core_axis_name="c", subcore_axis_name="s")

    buf = lambda shape, dt: [pltpu.VMEM(shape, dt) for _ in range(2)]
    sems = lambda n: [pltpu.SemaphoreType.DMA for _ in range(n)]

    @functools.partial(
        pl.kernel,
        mesh=mesh,
        out_type=jax.ShapeDtypeStruct((_HIST, 4, 32, 8, 128), jnp.float32),
        scratch_types=[
            buf((_SLAB, 128), jnp.int32),
            buf((_SLAB * 128, _EMBED), jnp.float32),
            buf((_SLAB * 128 * 33,), jnp.float32),
            buf((4, _SLAB, 8, 128), jnp.float32),
            sems(2), sems(2), sems(2),
        ],
        compiler_params=pltpu.CompilerParams(
            use_tc_tiling_on_sc=False, needs_layout_passes=False),
    )
    def gather_kernel(idx_hbm, table_hbm, out_hbm, idx_v, rows_v, pad_v,
                      out_v, isem, gsem, osem):
        w = lax.axis_index("s") * _NC + lax.axis_index("c")
        iota = lax.iota(jnp.int32, 16)
        iota33 = iota * 33

        def blk_coords(bid):
            h = bid // 8
            tb0 = (bid % 8) * _SLAB
            return h, h // 8, h % 8, tb0

        def fire_idx(bid, p, sync):
            h, th, k, tb0 = blk_coords(bid)
            for j in range(_SLAB):
                if sync:
                    pltpu.sync_copy(idx_hbm.at[th, tb0 + j, k], idx_v[p].at[j])
                else:
                    pltpu.make_async_copy(
                        idx_hbm.at[th, tb0 + j, k], idx_v[p].at[j],
                        isem[p]).start()

        def drain_idx(p):
            for j in range(_SLAB):
                pltpu.make_async_copy(
                    idx_hbm.at[0, j, 0], idx_v[p].at[j], isem[p]).wait()

        def fire_gather(p):
            for j in range(_SLAB):
                pltpu.make_async_copy(
                    table_hbm.at[idx_v[p].at[j]],
                    rows_v[p].at[pl.ds(j * 128, 128)],
                    gsem[p]).start()

        def drain_gather(p):
            for j in range(_SLAB):
                pltpu.make_async_copy(
                    table_hbm.at[idx_v[p].at[j]],
                    rows_v[p].at[pl.ds(j * 128, 128)],
                    gsem[p]).wait()

        def fire_out(bid, p):
            h, th, k, tb0 = blk_coords(bid)
            for i in range(4):
                pltpu.make_async_copy(
                    out_v[p].at[i], out_hbm.at[h, i, pl.ds(tb0, _SLAB)],
                    osem[p]).start()

        def drain_out(p):
            for i in range(4):
                pltpu.make_async_copy(
                    out_v[p].at[i], out_hbm.at[0, i, pl.ds(0, _SLAB)],
                    osem[p]).wait()

        def transpose(p):
            @functools.partial(plsc.parallel_loop, 0, 128, unroll=4)
            def _(r0):
                for rr in range(4):
                    r = r0 * 4 + rr
                    base = r * 33
                    pad_v[p][pl.ds(base, 16)] = rows_v[p][r, pl.ds(0, 16)]
                    pad_v[p][pl.ds(base + 16, 16)] = rows_v[p][r, pl.ds(16, 16)]

            @functools.partial(plsc.parallel_loop, 0, 64, unroll=2)
            def _(t):
                k2 = t // 8
                lbase = (t % 8) * 16
                for j in range(_SLAB):
                    base = (j * 128 + lbase) * 33 + k2
                    for i in range(4):
                        ids = (base + 8 * i) + iota33
                        vals = plsc.load_gather(pad_v[p], [ids])
                        out_v[p][i, j, k2, pl.ds(lbase, 16)] = vals

        for p in range(2):
            fire_idx(w * _BPW + p, p, sync=True)
            fire_gather(p)

        def body(g, carry):
            for p in range(2):
                bid = w * _BPW + 2 * g + p
                nxt = bid + 2
                drain_gather(p)

                @pl.when(2 * g + p + 2 < _BPW)
                def _():
                    fire_idx(nxt, p, sync=False)

                @pl.when(g > 0)
                def _():
                    drain_out(p)

                transpose(p)
                fire_out(bid, p)

                @pl.when(2 * g + p + 2 < _BPW)
                def _():
                    drain_idx(p)
                    fire_gather(p)
            return carry

        lax.fori_loop(0, _BPW // 2, body, 0)
        for p in range(2):
            drain_out(p)

    return gather_kernel


_gather = _build()


@jax.jit
def kernel(input_variable, table):
    idx_n = (input_variable.astype(jnp.int32).T
             .reshape(25, 8, 32, 128).transpose(0, 2, 1, 3))
    n = _gather(idx_n, table)
    return n.transpose(2, 4, 0, 1, 3).reshape(_BATCH, _HIST, _EMBED)
